# baseline (speedup 1.0000x reference)
import functools

import jax
import jax.numpy as jnp
from jax import lax
from jax.experimental import pallas as pl
from jax.experimental.pallas import tpu as pltpu
from jax.experimental.pallas import tpu_sc as plsc

VOCAB = 1000000
EMBED = 64
BATCH = 4096
HIST = 200

_NC = 2
_NS = 16
_NW = _NC * _NS
_BPW = BATCH // _NW
_PCOL = 32768


def _pack_body(t_ref, eye_ref, o_ref):
    o_ref[:, 0:EMBED] = t_ref[...].T


def _pack_table(tableT):
    grid = (pl.cdiv(VOCAB, _PCOL),)
    eye = jnp.eye(EMBED, dtype=jnp.float32)
    return pl.pallas_call(
        _pack_body,
        grid=grid,
        in_specs=[pl.BlockSpec((EMBED, _PCOL), lambda i: (0, i)),
                  pl.BlockSpec((EMBED, EMBED), lambda i: (0, 0))],
        out_specs=pl.BlockSpec((_PCOL, 2 * EMBED), lambda i: (i, 0)),
        out_shape=jax.ShapeDtypeStruct((VOCAB, 2 * EMBED), jnp.float32),
    )(tableT, eye)


def _sc_gather_pool(tableP, x):
    mesh = plsc.VectorSubcoreMesh(core_axis_name="c", subcore_axis_name="s")

    @functools.partial(
        pl.kernel,
        out_type=jax.ShapeDtypeStruct((BATCH, 2 * EMBED), jnp.float32),
        mesh=mesh,
        scratch_types=[
            pltpu.VMEM((_BPW, HIST), jnp.int32),
            pltpu.VMEM((HIST, _BPW), jnp.int32),
            pltpu.VMEM((_BPW, 2 * EMBED), jnp.float32),
            pltpu.SemaphoreType.DMA,
        ],
        compiler_params=pltpu.CompilerParams(use_tc_tiling_on_sc=False,
                                             needs_layout_passes=False),
    )
    def k(table_hbm, x_hbm, out_hbm, x_v, xt_v, pool_v, sem):
        wid = lax.axis_index("s") * _NC + lax.axis_index("c")
        pltpu.sync_copy(x_hbm.at[pl.ds(wid * _BPW, _BPW)], x_v)

        z = jnp.zeros((16,), jnp.float32)

        def zero_row(i, _):
            for c in range(8):
                pool_v[i, pl.ds(16 * c, 16)] = z
            return 0

        lax.fori_loop(0, _BPW, zero_row, 0)

        lanes = lax.iota(jnp.int32, 16)

        def tj(j, _):
            jv = jnp.full((16,), j, jnp.int32)
            for g in range(8):
                iv = plsc.load_gather(x_v, [16 * g + lanes, jv])
                xt_v[j, pl.ds(16 * g, 16)] = iv

            @pl.when(j > 0)
            def _():
                pltpu.async_copy(table_hbm.at[xt_v.at[j - 1]], pool_v, sem,
                                 add=True)
            return 0

        lax.fori_loop(0, HIST, tj, 0)
        pltpu.async_copy(table_hbm.at[xt_v.at[HIST - 1]], pool_v, sem,
                         add=True)

        def drain(j, _):
            pltpu.make_async_copy(table_hbm.at[xt_v.at[0]], pool_v, sem).wait()
            return 0

        lax.fori_loop(0, HIST, drain, 0)

        pltpu.sync_copy(pool_v, out_hbm.at[pl.ds(wid * _BPW, _BPW)])

    return k(tableP, x)


def _tc_finish_body(p_ref, wt_ref, b_ref, bng_ref, bnb_ref, lng_ref,
                    lnb_ref, o_ref):
    eps = 1e-5
    p = p_ref[:, 0:EMBED] * jnp.float32(1.0 / HIST)
    h = jnp.dot(p, wt_ref[...], preferred_element_type=jnp.float32)
    h = h + b_ref[...]
    mu = jnp.mean(h, axis=0, keepdims=True)
    var = jnp.mean((h - mu) ** 2, axis=0, keepdims=True)
    hb = (h - mu) / jnp.sqrt(var + eps) * bng_ref[...] + bnb_ref[...]
    lmu = jnp.mean(hb, axis=1, keepdims=True)
    lvar = jnp.mean((hb - lmu) ** 2, axis=1, keepdims=True)
    o_ref[...] = (hb - lmu) / jnp.sqrt(lvar + eps) * lng_ref[...] + lnb_ref[...]


def _tc_finish(pooled, Wt, b, bn_gamma, bn_beta, ln_gamma, ln_beta):
    return pl.pallas_call(
        _tc_finish_body,
        out_shape=jax.ShapeDtypeStruct((BATCH, EMBED), jnp.float32),
    )(pooled, Wt, b, bn_gamma, bn_beta, ln_gamma, ln_beta)


def kernel(x, table, W, b, bn_gamma, bn_beta, ln_gamma, ln_beta):
    x = x.astype(jnp.int32)
    tableP = _pack_table(table.T)
    pooled = _sc_gather_pool(tableP, x)
    return _tc_finish(
        pooled, W.T, b.reshape(1, EMBED),
        bn_gamma.reshape(1, EMBED), bn_beta.reshape(1, EMBED),
        ln_gamma.reshape(1, EMBED), ln_beta.reshape(1, EMBED))


# speedup vs baseline: 1.0024x; 1.0024x over previous
import functools

import jax
import jax.numpy as jnp
from jax import lax
from jax.experimental import pallas as pl
from jax.experimental.pallas import tpu as pltpu
from jax.experimental.pallas import tpu_sc as plsc

VOCAB = 1000000
EMBED = 64
BATCH = 4096
HIST = 200

_NC = 2
_NS = 16
_NW = _NC * _NS
_BPW = BATCH // _NW
_PCOL = 32768


def _pack_body(t_ref, o_ref):
    o_ref[:, 0:EMBED] = t_ref[...].T


def _pack_table(tableT):
    grid = (pl.cdiv(VOCAB, _PCOL),)
    return pl.pallas_call(
        _pack_body,
        grid=grid,
        in_specs=[pl.BlockSpec((EMBED, _PCOL), lambda i: (0, i))],
        out_specs=pl.BlockSpec((_PCOL, 2 * EMBED), lambda i: (i, 0)),
        out_shape=jax.ShapeDtypeStruct((VOCAB, 2 * EMBED), jnp.float32),
    )(tableT)


def _sc_gather_pool(tableP, x):
    mesh = plsc.VectorSubcoreMesh(core_axis_name="c", subcore_axis_name="s")

    @functools.partial(
        pl.kernel,
        out_type=jax.ShapeDtypeStruct((BATCH, 2 * EMBED), jnp.float32),
        mesh=mesh,
        scratch_types=[
            pltpu.VMEM((_BPW, HIST), jnp.int32),
            pltpu.VMEM((HIST, _BPW), jnp.int32),
            pltpu.VMEM((_BPW, 2 * EMBED), jnp.float32),
            pltpu.SemaphoreType.DMA,
        ],
        compiler_params=pltpu.CompilerParams(use_tc_tiling_on_sc=False,
                                             needs_layout_passes=False),
    )
    def k(table_hbm, x_hbm, out_hbm, x_v, xt_v, pool_v, sem):
        wid = lax.axis_index("s") * _NC + lax.axis_index("c")
        pltpu.sync_copy(x_hbm.at[pl.ds(wid * _BPW, _BPW)], x_v)

        z = jnp.zeros((16,), jnp.float32)

        def zero_row(i, _):
            for c in range(8):
                pool_v[i, pl.ds(16 * c, 16)] = z
            return 0

        lax.fori_loop(0, _BPW, zero_row, 0)

        lanes = lax.iota(jnp.int32, 16)

        def tj(j, _):
            jv = jnp.full((16,), j, jnp.int32)
            for g in range(8):
                iv = plsc.load_gather(x_v, [16 * g + lanes, jv])
                xt_v[j, pl.ds(16 * g, 16)] = iv

            @pl.when(j > 0)
            def _():
                pltpu.async_copy(table_hbm.at[xt_v.at[j - 1]], pool_v, sem,
                                 add=True)
            return 0

        lax.fori_loop(0, HIST, tj, 0)
        pltpu.async_copy(table_hbm.at[xt_v.at[HIST - 1]], pool_v, sem,
                         add=True)

        def drain(j, _):
            pltpu.make_async_copy(table_hbm.at[xt_v.at[0]], pool_v, sem).wait()
            return 0

        lax.fori_loop(0, HIST, drain, 0)

        pltpu.sync_copy(pool_v, out_hbm.at[pl.ds(wid * _BPW, _BPW)])

    return k(tableP, x)


def _tc_finish_body(p_ref, wt_ref, b_ref, bng_ref, bnb_ref, lng_ref,
                    lnb_ref, o_ref):
    eps = 1e-5
    p = p_ref[:, 0:EMBED] * jnp.float32(1.0 / HIST)
    h = jnp.dot(p, wt_ref[...], preferred_element_type=jnp.float32)
    h = h + b_ref[...]
    mu = jnp.mean(h, axis=0, keepdims=True)
    var = jnp.mean((h - mu) ** 2, axis=0, keepdims=True)
    hb = (h - mu) / jnp.sqrt(var + eps) * bng_ref[...] + bnb_ref[...]
    lmu = jnp.mean(hb, axis=1, keepdims=True)
    lvar = jnp.mean((hb - lmu) ** 2, axis=1, keepdims=True)
    o_ref[...] = (hb - lmu) / jnp.sqrt(lvar + eps) * lng_ref[...] + lnb_ref[...]


def _tc_finish(pooled, Wt, b, bn_gamma, bn_beta, ln_gamma, ln_beta):
    return pl.pallas_call(
        _tc_finish_body,
        out_shape=jax.ShapeDtypeStruct((BATCH, EMBED), jnp.float32),
    )(pooled, Wt, b, bn_gamma, bn_beta, ln_gamma, ln_beta)


def kernel(x, table, W, b, bn_gamma, bn_beta, ln_gamma, ln_beta):
    x = x.astype(jnp.int32)
    tableP = _pack_table(table.T)
    pooled = _sc_gather_pool(tableP, x)
    return _tc_finish(
        pooled, W.T, b.reshape(1, EMBED),
        bn_gamma.reshape(1, EMBED), bn_beta.reshape(1, EMBED),
        ln_gamma.reshape(1, EMBED), ln_beta.reshape(1, EMBED))
